# baseline (speedup 1.0000x reference)
import functools

import jax
import jax.numpy as jnp
from jax import lax
from jax.experimental import pallas as pl
from jax.experimental.pallas import tpu as pltpu
from jax.experimental.pallas import tpu_sc as plsc

N_NODES = 10000
N_EDGES = 320000
D = 128
NUM_SC = 2
NUM_TILES = 16
NUM_WORKERS = NUM_SC * NUM_TILES
E_PER_TILE = N_EDGES // NUM_WORKERS
CHUNK = 80
NCHUNK = 128
E_PAD = NCHUNK * CHUNK
SUPER = 4
NSUPER = NCHUNK // SUPER
N_PAD = 10240
ROWS_PER_TILE = N_PAD // NUM_TILES

_mesh = plsc.VectorSubcoreMesh(
    core_axis_name="c", subcore_axis_name="s",
    num_cores=NUM_SC, num_subcores=NUM_TILES,
)


@functools.partial(
    pl.kernel,
    out_type=jax.ShapeDtypeStruct((NUM_SC, N_PAD, D), jnp.float32),
    mesh=_mesh,
    scratch_types=[
        [pltpu.VMEM((SUPER, CHUNK), jnp.int32) for _ in range(2)],
        [pltpu.VMEM((SUPER * CHUNK,), jnp.int32) for _ in range(2)],
        [pltpu.VMEM((SUPER * CHUNK,), jnp.float32) for _ in range(2)],
        [pltpu.VMEM((CHUNK, D), jnp.float32) for _ in range(4)],
        pltpu.VMEM_SHARED((N_PAD, D), jnp.float32),
        [pltpu.SemaphoreType.DMA for _ in range(2)],
        [pltpu.SemaphoreType.DMA for _ in range(4)],
        [pltpu.SemaphoreType.DMA for _ in range(4)],
    ],
)
def _spmm_sc(x_hbm, row_hbm, col_hbm, val_hbm, zero_hbm, out_hbm,
             rbufs, cbufs, vbufs, fbufs, acc, esems, gsems, ssems):
    c = lax.axis_index("c")
    s = lax.axis_index("s")
    wid = c * NUM_TILES + s
    base = wid * E_PAD

    pltpu.sync_copy(zero_hbm, acc.at[pl.ds(s * ROWS_PER_TILE, ROWS_PER_TILE)])
    plsc.subcore_barrier()

    def issue_edges(sup, rot):
        sl = pl.ds(base + sup * (SUPER * CHUNK), SUPER * CHUNK)
        pltpu.async_copy(row_hbm.at[wid, sup], rbufs[rot], esems[rot])
        pltpu.async_copy(col_hbm.at[sl], cbufs[rot], esems[rot])
        pltpu.async_copy(val_hbm.at[sl], vbufs[rot], esems[rot])

    def wait_edges(rot):
        sl = pl.ds(base, SUPER * CHUNK)
        pltpu.make_async_copy(row_hbm.at[wid, 0], rbufs[rot], esems[rot]).wait()
        pltpu.make_async_copy(col_hbm.at[sl], cbufs[rot], esems[rot]).wait()
        pltpu.make_async_copy(val_hbm.at[sl], vbufs[rot], esems[rot]).wait()

    def issue_gather(k, rot, q):
        pltpu.async_copy(
            x_hbm.at[cbufs[rot].at[pl.ds(q * CHUNK, CHUNK)]],
            fbufs[k], gsems[k])

    def wait_gather(k, rot, q):
        pltpu.make_async_copy(
            x_hbm.at[cbufs[rot].at[pl.ds(q * CHUNK, CHUNK)]],
            fbufs[k], gsems[k]).wait()

    def issue_scatter(k, rot, q):
        pltpu.async_copy(
            fbufs[k], acc.at[rbufs[rot].at[q]], ssems[k], add=True)

    def wait_scatter(k, rot, q):
        pltpu.make_async_copy(
            fbufs[k], acc.at[rbufs[rot].at[q]], ssems[k]).wait()

    def scale(k, rot, q):
        buf, vals = fbufs[k], vbufs[rot]

        @pl.loop(0, CHUNK // 16)
        def _grp(g):
            vv = vals[pl.ds(q * CHUNK + g * 16, 16)]

            @pl.loop(0, 16, unroll=4)
            def _row(r2):
                vb = vv.at[jnp.full((16,), r2, jnp.int32)].get(
                    mode="promise_in_bounds")
                r = g * 16 + r2
                for qq in range(D // 16):
                    buf[r, pl.ds(qq * 16, 16)] = (
                        buf[r, pl.ds(qq * 16, 16)] * vb)

    def chunk_body(j, ph, *, ws=True, ie=True, we=True, ig=True):
        q = ph % 4
        sp = (ph >> 2) % 2
        if ws:
            wait_scatter((ph + 1) % 4, ((ph - 3) >> 2) % 2, (ph - 3) % 4)
        if ie and q == 2:
            issue_edges((j >> 2) + 1, (sp + 1) % 2)
        if we and q == 3:
            wait_edges((sp + 1) % 2)
        if ig:
            issue_gather((ph + 1) % 4, ((ph + 1) >> 2) % 2, (ph + 1) % 4)
        wait_gather(q, sp, q)
        scale(q, sp, q)
        issue_scatter(q, sp, q)

    issue_edges(0, 0)
    issue_edges(1, 1)
    wait_edges(0)
    issue_gather(0, 0, 0)
    chunk_body(0, 8, ws=False)
    chunk_body(1, 9, ws=False)
    chunk_body(2, 10, ws=False, ie=False)
    for t in range(3, 8):
        chunk_body(t, 8 + t)

    @pl.loop(8, NCHUNK - 8, step=8)
    def _eight(J):
        for t in range(8):
            chunk_body(J + t, 8 + t)

    for t in range(120, 128):
        chunk_body(t, 8 + t % 8,
                   ie=(t != 126), we=(t != 127), ig=(t != 127))
    wait_scatter(1, 1, 1)
    wait_scatter(2, 1, 2)
    wait_scatter(3, 1, 3)

    plsc.subcore_barrier()
    pltpu.sync_copy(acc.at[pl.ds(s * ROWS_PER_TILE, ROWS_PER_TILE)],
                    out_hbm.at[c, pl.ds(s * ROWS_PER_TILE, ROWS_PER_TILE)])


def _combine_tc(p_ref, w_ref, b_ref, o_ref):
    agg = p_ref[0] + p_ref[1]
    o_ref[...] = (
        jnp.dot(agg, w_ref[...], preferred_element_type=jnp.float32)
        + b_ref[...]
    )


_BLK_M = 2000


def kernel(x, adj_mat_indices, adj_mat_values, weights, bias):
    pad = ((0, 0), (0, E_PAD - E_PER_TILE))
    row = jnp.pad(adj_mat_indices[0].reshape(NUM_WORKERS, E_PER_TILE), pad)
    col = jnp.pad(adj_mat_indices[1].reshape(NUM_WORKERS, E_PER_TILE), pad)
    val = jnp.pad(adj_mat_values.reshape(NUM_WORKERS, E_PER_TILE), pad)
    row4 = row.reshape(NUM_WORKERS, NSUPER, SUPER, CHUNK)
    zero = jnp.zeros((ROWS_PER_TILE, D), jnp.float32)
    parts = _spmm_sc(x, row4, col.reshape(-1), val.reshape(-1), zero)
    return pl.pallas_call(
        _combine_tc,
        grid=(N_NODES // _BLK_M,),
        in_specs=[
            pl.BlockSpec((NUM_SC, _BLK_M, D), lambda i: (0, i, 0)),
            pl.BlockSpec((D, D), lambda i: (0, 0)),
            pl.BlockSpec((1, D), lambda i: (0, 0)),
        ],
        out_specs=pl.BlockSpec((_BLK_M, D), lambda i: (i, 0)),
        out_shape=jax.ShapeDtypeStruct((N_NODES, D), jnp.float32),
    )(parts, weights, bias.reshape(1, D))


# speedup vs baseline: 1.7434x; 1.7434x over previous
import functools

import jax
import jax.numpy as jnp
from jax import lax
from jax.experimental import pallas as pl
from jax.experimental.pallas import tpu as pltpu
from jax.experimental.pallas import tpu_sc as plsc

N_NODES = 10000
N_EDGES = 320000
D = 128
NUM_SC = 2
NUM_TILES = 16
NUM_WORKERS = NUM_SC * NUM_TILES
E_PER_TILE = N_EDGES // NUM_WORKERS
CHUNK = 80
NCHUNK = 126
E_PAD = NCHUNK * CHUNK
SUPER = 3
NSUPER = NCHUNK // SUPER
SE = SUPER * CHUNK
N_PAD = 10240
ROWS_PER_TILE = N_PAD // NUM_TILES

_mesh = plsc.VectorSubcoreMesh(
    core_axis_name="c", subcore_axis_name="s",
    num_cores=NUM_SC, num_subcores=NUM_TILES,
)


@functools.partial(
    pl.kernel,
    out_type=jax.ShapeDtypeStruct((NUM_SC, N_PAD, D), jnp.float32),
    mesh=_mesh,
    scratch_types=[
        pltpu.VMEM((NCHUNK, CHUNK), jnp.int32),
        [pltpu.VMEM((SE,), jnp.int32) for _ in range(2)],
        [pltpu.VMEM((SE,), jnp.float32) for _ in range(2)],
        [pltpu.VMEM((CHUNK, D), jnp.float32) for _ in range(3)],
        pltpu.VMEM_SHARED((N_PAD, D), jnp.float32),
        pltpu.SemaphoreType.DMA,
        [pltpu.SemaphoreType.DMA for _ in range(2)],
        [pltpu.SemaphoreType.DMA for _ in range(3)],
        [pltpu.SemaphoreType.DMA for _ in range(3)],
    ],
)
def _spmm_sc(x_hbm, row_hbm, col_hbm, val_hbm, zero_hbm, out_hbm,
             rowv, cbufs, vbufs, fbufs, acc, rsem, esems, gsems, ssems):
    c = lax.axis_index("c")
    s = lax.axis_index("s")
    wid = c * NUM_TILES + s
    base = wid * E_PAD

    pltpu.sync_copy(zero_hbm, acc.at[pl.ds(s * ROWS_PER_TILE, ROWS_PER_TILE)])
    plsc.subcore_barrier()

    def issue_cv(sup, rot):
        sl = pl.ds(base + sup * SE, SE)
        pltpu.async_copy(col_hbm.at[sl], cbufs[rot], esems[rot])
        pltpu.async_copy(val_hbm.at[sl], vbufs[rot], esems[rot])

    def wait_cv(rot):
        sl = pl.ds(base, SE)
        pltpu.make_async_copy(col_hbm.at[sl], cbufs[rot], esems[rot]).wait()
        pltpu.make_async_copy(val_hbm.at[sl], vbufs[rot], esems[rot]).wait()

    def issue_gather(k, rot, qs):
        pltpu.async_copy(
            x_hbm.at[cbufs[rot].at[pl.ds(qs * CHUNK, CHUNK)]],
            fbufs[k], gsems[k])

    def wait_gather(k, rot, qs):
        pltpu.make_async_copy(
            x_hbm.at[cbufs[rot].at[pl.ds(qs * CHUNK, CHUNK)]],
            fbufs[k], gsems[k]).wait()

    def issue_scatter(k, j):
        pltpu.async_copy(fbufs[k], acc.at[rowv.at[j]], ssems[k], add=True)

    def wait_scatter(k):
        pltpu.make_async_copy(fbufs[k], acc.at[rowv.at[0]], ssems[k]).wait()

    def scale(k, rot, qs):
        buf, vals = fbufs[k], vbufs[rot]

        @pl.loop(0, CHUNK // 16)
        def _grp(g):
            vv = vals[pl.ds(qs * CHUNK + g * 16, 16)]

            @pl.loop(0, 16, unroll=4)
            def _row(r2):
                vb = vv.at[jnp.full((16,), r2, jnp.int32)].get(
                    mode="promise_in_bounds")
                r = g * 16 + r2
                for q in range(D // 16):
                    buf[r, pl.ds(q * 16, 16)] = buf[r, pl.ds(q * 16, 16)] * vb

    def chunk_body(j, ph, *, ws=True, ie=True, we=True, ig=True):
        k = ph % 3
        rot = (ph // 3) % 2
        if ws:
            wait_scatter((ph + 1) % 3)
        if ie and k == 0:
            issue_cv(j // 3 + 1, (rot + 1) % 2)
        if we and k == 2:
            wait_cv((rot + 1) % 2)
        if ig:
            issue_gather((ph + 1) % 3, ((ph + 1) // 3) % 2, (ph + 1) % 3)
        wait_gather(k, rot, k)
        scale(k, rot, k)
        issue_scatter(k, j)

    pltpu.async_copy(row_hbm.at[wid], rowv, rsem)
    issue_cv(0, 0)
    issue_cv(1, 1)
    wait_cv(0)
    issue_gather(0, 0, 0)
    pltpu.make_async_copy(row_hbm.at[wid], rowv, rsem).wait()

    chunk_body(0, 6, ws=False, ie=False)
    chunk_body(1, 7, ws=False)
    for t in range(2, 6):
        chunk_body(t, 6 + t)

    @pl.loop(6, NCHUNK - 6, step=6)
    def _six(J):
        for t in range(6):
            chunk_body(J + t, 6 + t)

    for t in range(120, 126):
        chunk_body(t, 6 + t % 6,
                   ie=(t != 123), we=(t != 125), ig=(t != 125))
    wait_scatter((NCHUNK - 2) % 3)
    wait_scatter((NCHUNK - 1) % 3)

    plsc.subcore_barrier()
    pltpu.sync_copy(acc.at[pl.ds(s * ROWS_PER_TILE, ROWS_PER_TILE)],
                    out_hbm.at[c, pl.ds(s * ROWS_PER_TILE, ROWS_PER_TILE)])


def _combine_tc(p_ref, w_ref, b_ref, o_ref):
    agg = p_ref[0] + p_ref[1]
    o_ref[...] = (
        jnp.dot(agg, w_ref[...], preferred_element_type=jnp.float32)
        + b_ref[...]
    )


_BLK_M = 2000


def kernel(x, adj_mat_indices, adj_mat_values, weights, bias):
    pad = ((0, 0), (0, E_PAD - E_PER_TILE))
    row = jnp.pad(adj_mat_indices[0].reshape(NUM_WORKERS, E_PER_TILE), pad)
    col = jnp.pad(adj_mat_indices[1].reshape(NUM_WORKERS, E_PER_TILE), pad)
    val = jnp.pad(adj_mat_values.reshape(NUM_WORKERS, E_PER_TILE), pad)
    row3 = row.reshape(NUM_WORKERS, NCHUNK, CHUNK)
    zero = jnp.zeros((ROWS_PER_TILE, D), jnp.float32)
    parts = _spmm_sc(x, row3, col.reshape(-1), val.reshape(-1), zero)
    return pl.pallas_call(
        _combine_tc,
        grid=(N_NODES // _BLK_M,),
        in_specs=[
            pl.BlockSpec((NUM_SC, _BLK_M, D), lambda i: (0, i, 0)),
            pl.BlockSpec((D, D), lambda i: (0, 0)),
            pl.BlockSpec((1, D), lambda i: (0, 0)),
        ],
        out_specs=pl.BlockSpec((_BLK_M, D), lambda i: (i, 0)),
        out_shape=jax.ShapeDtypeStruct((N_NODES, D), jnp.float32),
    )(parts, weights, bias.reshape(1, D))
